# baseline (speedup 1.0000x reference)
import jax
import jax.numpy as jnp
from jax import lax
from jax.experimental import pallas as pl
from jax.experimental.pallas import tpu as pltpu
from jax.experimental.pallas import tpu_sc as plsc

N = 10000
E = 320000
D = 128
H = 128
C = 16

NC = 2
NS = 16
NW = NC * NS
CH = 64
SBC = 8
TOT_CHUNK = E // CH
TOT_SB = TOT_CHUNK // SBC
N_ACC = 10240
ROWS_PER_TILE = N_ACC // NS

_mesh = plsc.VectorSubcoreMesh(
    core_axis_name="c", subcore_axis_name="s", num_cores=NC, num_subcores=NS)

_f32 = jnp.float32
_i32 = jnp.int32


def _my_range(c, s):
    wid = c * NS + s
    base_sb = 20 * wid - jnp.maximum(wid - 17, 0)
    n_sb = jnp.where(wid < 17, 20, 19)
    return base_sb, n_sb


def _hist_body(row2, out_hbm, acc1d, idxb, ones_buf, zbuf):
    c = lax.axis_index("c")
    s = lax.axis_index("s")
    zero16 = jnp.zeros((16,), _f32)
    one16 = jnp.ones((16,), _f32)

    for k in range(CH // 16):
        ones_buf[pl.ds(k * 16, 16)] = one16

    def _zb(k, _):
        zbuf[pl.ds(k * 16, 16)] = zero16
        return 0
    lax.fori_loop(0, ROWS_PER_TILE // 16, _zb, 0)

    pltpu.sync_copy(zbuf, acc1d.at[pl.ds(s * ROWS_PER_TILE, ROWS_PER_TILE)])
    plsc.subcore_barrier()

    base_sb, n_sb = _my_range(c, s)

    def _sb(sb, _):
        pltpu.sync_copy(row2.at[pl.ds((base_sb + sb) * SBC, SBC)], idxb)

        def _sc(j, _):
            pltpu.sync_copy(ones_buf, acc1d.at[idxb.at[j]], add=True)
            return 0
        lax.fori_loop(0, SBC, _sc, 0)
        return 0
    lax.fori_loop(0, n_sb, _sb, 0)

    plsc.subcore_barrier()
    pltpu.sync_copy(acc1d.at[pl.ds(s * ROWS_PER_TILE, ROWS_PER_TILE)],
                    out_hbm.at[pl.ds(c * N_ACC + s * ROWS_PER_TILE, ROWS_PER_TILE)])


_hist = pl.kernel(
    _hist_body,
    out_type=jax.ShapeDtypeStruct((NC * N_ACC,), _f32),
    mesh=_mesh,
    scratch_types=[
        pltpu.VMEM_SHARED((N_ACC,), _f32),
        pltpu.VMEM((SBC, CH), _i32),
        pltpu.VMEM((CH,), _f32),
        pltpu.VMEM((ROWS_PER_TILE,), _f32),
    ],
)


def _agg_body(g_hbm, row2, col2, out_hbm, acc, idxr, idxc, db0, db1,
              gs0, gs1, ss0, ss1):
    c = lax.axis_index("c")
    s = lax.axis_index("s")
    zero16 = jnp.zeros((16,), _f32)

    def _zb(r, _):
        for k in range(8):
            db0[r, pl.ds(k * 16, 16)] = zero16
        return 0
    lax.fori_loop(0, CH, _zb, 0)
    base = s * ROWS_PER_TILE

    def _za(k, _):
        pltpu.sync_copy(db0, acc.at[pl.ds(base + k * CH, CH)])
        return 0
    lax.fori_loop(0, ROWS_PER_TILE // CH, _za, 0)

    plsc.subcore_barrier()

    base_sb, n_sb = _my_range(c, s)

    def _sb(sb, _):
        start = (base_sb + sb) * SBC
        pltpu.sync_copy(row2.at[pl.ds(start, SBC)], idxr)
        pltpu.sync_copy(col2.at[pl.ds(start, SBC)], idxc)
        pltpu.async_copy(g_hbm.at[idxc.at[0]], db0, gs0)
        pltpu.async_copy(g_hbm.at[idxc.at[1]], db1, gs1)

        def _step(p, _):
            j0 = p * 2
            pltpu.make_async_copy(g_hbm.at[idxc.at[j0]], db0, gs0).wait()
            pltpu.async_copy(db0, acc.at[idxr.at[j0]], ss0, add=True)
            pltpu.make_async_copy(g_hbm.at[idxc.at[j0 + 1]], db1, gs1).wait()
            pltpu.async_copy(db1, acc.at[idxr.at[j0 + 1]], ss1, add=True)

            @pl.when(p + 1 < SBC // 2)
            def _():
                pltpu.make_async_copy(db0, acc.at[idxr.at[j0]], ss0).wait()
                pltpu.async_copy(g_hbm.at[idxc.at[j0 + 2]], db0, gs0)
                pltpu.make_async_copy(db1, acc.at[idxr.at[j0 + 1]], ss1).wait()
                pltpu.async_copy(g_hbm.at[idxc.at[j0 + 3]], db1, gs1)
            return 0
        lax.fori_loop(0, SBC // 2, _step, 0)
        pltpu.make_async_copy(db0, acc.at[idxr.at[SBC - 2]], ss0).wait()
        pltpu.make_async_copy(db1, acc.at[idxr.at[SBC - 1]], ss1).wait()
        return 0
    lax.fori_loop(0, n_sb, _sb, 0)

    plsc.subcore_barrier()
    pltpu.sync_copy(acc.at[pl.ds(base, ROWS_PER_TILE)],
                    out_hbm.at[c, pl.ds(base, ROWS_PER_TILE)])


_agg = pl.kernel(
    _agg_body,
    out_type=jax.ShapeDtypeStruct((NC, N_ACC, 128), _f32),
    mesh=_mesh,
    scratch_types=[
        pltpu.VMEM_SHARED((N_ACC, 128), _f32),
        pltpu.VMEM((SBC, CH), _i32),
        pltpu.VMEM((SBC, CH), _i32),
        pltpu.VMEM((CH, 128), _f32),
        pltpu.VMEM((CH, 128), _f32),
        pltpu.SemaphoreType.DMA,
        pltpu.SemaphoreType.DMA,
        pltpu.SemaphoreType.DMA,
        pltpu.SemaphoreType.DMA,
    ],
)


_B = 2000


def _tc1_body(x_ref, c0_ref, c1_ref, w_ref, g_ref, dis_ref):
    deg = c0_ref[0] + c1_ref[0] + 5.0
    dis = lax.rsqrt(deg)
    h = jnp.dot(x_ref[...], w_ref[...], preferred_element_type=_f32)
    g_ref[...] = h * dis
    dis_ref[...] = dis


def _tc2_body(a_ref, b_ref, g1_ref, dis_ref, w_ref, g2_ref):
    x2 = jnp.maximum(
        dis_ref[...] * (a_ref[0] + b_ref[0] + 5.0 * g1_ref[...]), 0.0)
    h2 = jnp.dot(x2, w_ref[...], preferred_element_type=_f32)
    g2_ref[...] = h2 * dis_ref[...]


def _tc3_body(a_ref, b_ref, g2_ref, dis_ref, w_ref, b3_ref, o_ref):
    hf = dis_ref[...] * (a_ref[0] + b_ref[0] + 5.0 * g2_ref[...])
    o_ref[...] = jnp.dot(hf, w_ref[...], preferred_element_type=_f32) + b3_ref[...]


def _row_blk(last):
    return pl.BlockSpec((_B, last), lambda i: (i, 0))


def _plane_blk(plane, last):
    return pl.BlockSpec((1, _B, last), lambda i, p=plane: (p, i, 0))


def _full(shape):
    return pl.BlockSpec(shape, lambda i: tuple(0 for _ in shape))


_tc1 = pl.pallas_call(
    _tc1_body,
    grid=(N // _B,),
    in_specs=[_row_blk(D), _plane_blk(0, 1), _plane_blk(1, 1), _full((D, H))],
    out_specs=[_row_blk(H), _row_blk(1)],
    out_shape=[jax.ShapeDtypeStruct((N, H), _f32),
               jax.ShapeDtypeStruct((N, 1), _f32)],
)

_tc2 = pl.pallas_call(
    _tc2_body,
    grid=(N // _B,),
    in_specs=[_plane_blk(0, 128), _plane_blk(1, 128), _row_blk(H), _row_blk(1),
              _full((H, H))],
    out_specs=_row_blk(H),
    out_shape=jax.ShapeDtypeStruct((N, H), _f32),
)

_tc3 = pl.pallas_call(
    _tc3_body,
    grid=(N // _B,),
    in_specs=[_plane_blk(0, 128), _plane_blk(1, 128), _row_blk(H), _row_blk(1),
              _full((H, C)), _full((1, C))],
    out_specs=_row_blk(C),
    out_shape=jax.ShapeDtypeStruct((N, C), _f32),
)


@jax.jit
def kernel(x, edge_index, W1, W2, W3, b3):
    row2 = edge_index[0].reshape(TOT_CHUNK, CH)
    col2 = edge_index[1].reshape(TOT_CHUNK, CH)

    counts = _hist(row2).reshape(NC, N_ACC, 1)

    g1, dis = _tc1(x, counts, counts, W1)
    agg1 = _agg(g1, row2, col2)
    g2 = _tc2(agg1, agg1, g1, dis, W2)
    agg2 = _agg(g2, row2, col2)
    return _tc3(agg2, agg2, g2, dis, W3, b3.reshape(1, C))


# speedup vs baseline: 1.1294x; 1.1294x over previous
import jax
import jax.numpy as jnp
from jax import lax
from jax.experimental import pallas as pl
from jax.experimental.pallas import tpu as pltpu
from jax.experimental.pallas import tpu_sc as plsc

N = 10000
E = 320000
D = 128
H = 128
C = 16

NC = 2
NS = 16
NW = NC * NS
CH = 64
SBC = 8
TOT_CHUNK = E // CH
TOT_SB = TOT_CHUNK // SBC
N_ACC = 10240
ROWS_PER_TILE = N_ACC // NS

_mesh = plsc.VectorSubcoreMesh(
    core_axis_name="c", subcore_axis_name="s", num_cores=NC, num_subcores=NS)

_f32 = jnp.float32
_i32 = jnp.int32


def _my_range(c, s):
    wid = c * NS + s
    base_sb = 20 * wid - jnp.maximum(wid - 17, 0)
    n_sb = jnp.where(wid < 17, 20, 19)
    return base_sb, n_sb


def _hist_body(row2, out_hbm, acc1d, idxb, ones_buf, zbuf):
    c = lax.axis_index("c")
    s = lax.axis_index("s")
    zero16 = jnp.zeros((16,), _f32)
    one16 = jnp.ones((16,), _f32)

    for k in range(CH // 16):
        ones_buf[pl.ds(k * 16, 16)] = one16

    def _zb(k, _):
        zbuf[pl.ds(k * 16, 16)] = zero16
        return 0
    lax.fori_loop(0, ROWS_PER_TILE // 16, _zb, 0)

    pltpu.sync_copy(zbuf, acc1d.at[pl.ds(s * ROWS_PER_TILE, ROWS_PER_TILE)])

    base_sb, n_sb = _my_range(c, s)
    base_ch = base_sb * SBC
    wid = c * NS + s
    pltpu.sync_copy(row2.at[pl.ds(base_ch, 152)], idxb.at[pl.ds(0, 152)])

    @pl.when(wid < 17)
    def _():
        pltpu.sync_copy(row2.at[pl.ds(base_ch + 152, 8)],
                        idxb.at[pl.ds(152, 8)])
    plsc.subcore_barrier()

    def _sc(j, _):
        pltpu.sync_copy(ones_buf, acc1d.at[idxb.at[j]], add=True)
        return 0
    lax.fori_loop(0, n_sb * SBC, _sc, 0)

    plsc.subcore_barrier()
    pltpu.sync_copy(acc1d.at[pl.ds(s * ROWS_PER_TILE, ROWS_PER_TILE)],
                    out_hbm.at[pl.ds(c * N_ACC + s * ROWS_PER_TILE, ROWS_PER_TILE)])


_hist = pl.kernel(
    _hist_body,
    out_type=jax.ShapeDtypeStruct((NC * N_ACC,), _f32),
    mesh=_mesh,
    scratch_types=[
        pltpu.VMEM_SHARED((N_ACC,), _f32),
        pltpu.VMEM((160, CH), _i32),
        pltpu.VMEM((CH,), _f32),
        pltpu.VMEM((ROWS_PER_TILE,), _f32),
    ],
)


def _agg_body(g_hbm, row2, col2, out_hbm, acc, idxr, idxc, db0, db1,
              gs0, gs1, ss0, ss1):
    c = lax.axis_index("c")
    s = lax.axis_index("s")
    zero16 = jnp.zeros((16,), _f32)

    def _zb(r, _):
        for k in range(8):
            db0[r, pl.ds(k * 16, 16)] = zero16
        return 0
    lax.fori_loop(0, CH, _zb, 0)
    base = s * ROWS_PER_TILE

    def _za(k, _):
        pltpu.sync_copy(db0, acc.at[pl.ds(base + k * CH, CH)])
        return 0
    lax.fori_loop(0, ROWS_PER_TILE // CH, _za, 0)

    plsc.subcore_barrier()

    base_sb, n_sb = _my_range(c, s)

    def _sb(sb, _):
        start = (base_sb + sb) * SBC
        pltpu.sync_copy(row2.at[pl.ds(start, SBC)], idxr)
        pltpu.sync_copy(col2.at[pl.ds(start, SBC)], idxc)
        pltpu.async_copy(g_hbm.at[idxc.at[0]], db0, gs0)

        def _step(p, _):
            j0 = p * 2
            pltpu.async_copy(g_hbm.at[idxc.at[j0 + 1]], db1, gs1)
            pltpu.make_async_copy(g_hbm.at[idxc.at[j0]], db0, gs0).wait()
            pltpu.sync_copy(db0, acc.at[idxr.at[j0]], add=True)

            @pl.when(p + 1 < SBC // 2)
            def _():
                pltpu.async_copy(g_hbm.at[idxc.at[j0 + 2]], db0, gs0)
            pltpu.make_async_copy(g_hbm.at[idxc.at[j0 + 1]], db1, gs1).wait()
            pltpu.sync_copy(db1, acc.at[idxr.at[j0 + 1]], add=True)
            return 0
        lax.fori_loop(0, SBC // 2, _step, 0)
        return 0
    lax.fori_loop(0, n_sb, _sb, 0)

    plsc.subcore_barrier()
    pltpu.sync_copy(acc.at[pl.ds(base, ROWS_PER_TILE)],
                    out_hbm.at[c, pl.ds(base, ROWS_PER_TILE)])


_agg = pl.kernel(
    _agg_body,
    out_type=jax.ShapeDtypeStruct((NC, N_ACC, 128), _f32),
    mesh=_mesh,
    scratch_types=[
        pltpu.VMEM_SHARED((N_ACC, 128), _f32),
        pltpu.VMEM((SBC, CH), _i32),
        pltpu.VMEM((SBC, CH), _i32),
        pltpu.VMEM((CH, 128), _f32),
        pltpu.VMEM((CH, 128), _f32),
        pltpu.SemaphoreType.DMA,
        pltpu.SemaphoreType.DMA,
        pltpu.SemaphoreType.DMA,
        pltpu.SemaphoreType.DMA,
    ],
)


_B = 2000


def _tc1_body(x_ref, c0_ref, c1_ref, w_ref, g_ref, dis_ref):
    deg = c0_ref[0] + c1_ref[0] + 5.0
    dis = lax.rsqrt(deg)
    h = jnp.dot(x_ref[...], w_ref[...], preferred_element_type=_f32)
    g_ref[...] = h * dis
    dis_ref[...] = dis


def _tc2_body(a_ref, b_ref, g1_ref, dis_ref, w_ref, g2_ref):
    x2 = jnp.maximum(
        dis_ref[...] * (a_ref[0] + b_ref[0] + 5.0 * g1_ref[...]), 0.0)
    h2 = jnp.dot(x2, w_ref[...], preferred_element_type=_f32)
    g2_ref[...] = h2 * dis_ref[...]


def _tc3_body(a_ref, b_ref, g2_ref, dis_ref, w_ref, b3_ref, o_ref):
    hf = dis_ref[...] * (a_ref[0] + b_ref[0] + 5.0 * g2_ref[...])
    o_ref[...] = jnp.dot(hf, w_ref[...], preferred_element_type=_f32) + b3_ref[...]


def _row_blk(last):
    return pl.BlockSpec((_B, last), lambda i: (i, 0))


def _plane_blk(plane, last):
    return pl.BlockSpec((1, _B, last), lambda i, p=plane: (p, i, 0))


def _full(shape):
    return pl.BlockSpec(shape, lambda i: tuple(0 for _ in shape))


_tc1 = pl.pallas_call(
    _tc1_body,
    grid=(N // _B,),
    in_specs=[_row_blk(D), _plane_blk(0, 1), _plane_blk(1, 1), _full((D, H))],
    out_specs=[_row_blk(H), _row_blk(1)],
    out_shape=[jax.ShapeDtypeStruct((N, H), _f32),
               jax.ShapeDtypeStruct((N, 1), _f32)],
)

_tc2 = pl.pallas_call(
    _tc2_body,
    grid=(N // _B,),
    in_specs=[_plane_blk(0, 128), _plane_blk(1, 128), _row_blk(H), _row_blk(1),
              _full((H, H))],
    out_specs=_row_blk(H),
    out_shape=jax.ShapeDtypeStruct((N, H), _f32),
)

_tc3 = pl.pallas_call(
    _tc3_body,
    grid=(N // _B,),
    in_specs=[_plane_blk(0, 128), _plane_blk(1, 128), _row_blk(H), _row_blk(1),
              _full((H, C)), _full((1, C))],
    out_specs=_row_blk(C),
    out_shape=jax.ShapeDtypeStruct((N, C), _f32),
)


@jax.jit
def kernel(x, edge_index, W1, W2, W3, b3):
    row2 = edge_index[0].reshape(TOT_CHUNK, CH)
    col2 = edge_index[1].reshape(TOT_CHUNK, CH)

    counts = _hist(row2).reshape(NC, N_ACC, 1)

    g1, dis = _tc1(x, counts, counts, W1)
    agg1 = _agg(g1, row2, col2)
    g2 = _tc2(agg1, agg1, g1, dis, W2)
    agg2 = _agg(g2, row2, col2)
    return _tc3(agg2, agg2, g2, dis, W3, b3.reshape(1, C))
